# baseline (speedup 1.0000x reference)
import functools

import jax
import jax.numpy as jnp
from jax import lax
from jax.experimental import pallas as pl
from jax.experimental.pallas import tpu as pltpu
from jax.experimental.pallas import tpu_sc as plsc

EMBED = 64
PADDED = 128
NW = 32
CHUNK = 128


@functools.cache
def _make_gather(total: int, vocab: int):
    per_w = total // NW
    nchunk = per_w // CHUNK
    mesh = plsc.VectorSubcoreMesh(core_axis_name="c", subcore_axis_name="s")

    @functools.partial(
        pl.kernel,
        mesh=mesh,
        out_type=jax.ShapeDtypeStruct((total, PADDED), jnp.float32),
        scratch_types=[
            pltpu.VMEM((nchunk, CHUNK), jnp.int32),
            pltpu.VMEM((2, CHUNK, PADDED), jnp.float32),
            pltpu.SemaphoreType.DMA,
            pltpu.SemaphoreType.DMA,
        ],
        compiler_params=pltpu.CompilerParams(use_tc_tiling_on_sc=True),
    )
    def gather(idx_hbm, table_hbm, out_hbm, idx_v, rows_v, gsem, osem):
        wid = lax.axis_index("s") * 2 + lax.axis_index("c")
        base = wid * per_w
        pltpu.sync_copy(idx_hbm.at[wid], idx_v)

        def fire_gather(j, buf):
            pltpu.async_copy(table_hbm.at[idx_v.at[j]], rows_v.at[buf], gsem)

        def wait_gather(j, buf):
            pltpu.make_async_copy(
                table_hbm.at[idx_v.at[j]], rows_v.at[buf], gsem).wait()

        def fire_outcopy(j, buf):
            pltpu.async_copy(
                rows_v.at[buf], out_hbm.at[pl.ds(base + j * CHUNK, CHUNK)], osem)

        def wait_outcopy(buf):
            pltpu.make_async_copy(
                rows_v.at[buf], out_hbm.at[pl.ds(base, CHUNK)], osem).wait()

        fire_gather(0, 0)

        def body(j, carry):
            buf = lax.rem(j, 2)
            wait_gather(j, buf)

            @pl.when(j >= 1)
            def _():
                wait_outcopy(1 - buf)

            fire_outcopy(j, buf)

            @pl.when(j + 1 < nchunk)
            def _():
                fire_gather(j + 1, 1 - buf)
            return carry

        lax.fori_loop(0, nchunk, body, 0, unroll=False)
        wait_outcopy(lax.rem(nchunk - 1, 2))

    return gather


def kernel(x, word_embed):
    batch, hist = x.shape
    total = batch * hist
    idx3d = x.astype(jnp.int32).reshape(NW, total // (NW * CHUNK), CHUNK)
    t128 = jnp.pad(word_embed, ((0, 0), (0, PADDED - EMBED)))
    out2 = _make_gather(total, word_embed.shape[0])(idx3d, t128)
    return out2[:, :EMBED].reshape(batch, hist, EMBED)
